# baseline (device time: 968208 ns/iter reference)
import jax
import jax.numpy as jnp
from jax import lax
from jax.experimental import pallas as pl
from jax.experimental.pallas import tpu as pltpu

T_LOCAL = 4096
D = 2048
F = 4096
E_LOCAL = 4
TILE_F = 512
CAP = 1152
S_TILE = 512
K_TILE = 512
N_SLOTS = E_LOCAL * CAP


def _assign_xchg_body(a_ref, aall_ref, sems):
    my_x = lax.axis_index("x")
    my_y = lax.axis_index("y")
    peer = (my_x, 1 - my_y)

    barrier = pltpu.get_barrier_semaphore()
    pl.semaphore_signal(
        barrier, inc=1, device_id=peer, device_id_type=pl.DeviceIdType.MESH
    )
    pl.semaphore_wait(barrier, 1)

    aall_ref[pl.ds(my_y, 1), :] = a_ref[...]
    rdma = pltpu.make_async_remote_copy(
        src_ref=a_ref,
        dst_ref=aall_ref.at[pl.ds(my_y, 1), :],
        send_sem=sems.at[0],
        recv_sem=sems.at[1],
        device_id=peer,
        device_id_type=pl.DeviceIdType.MESH,
    )
    rdma.start()
    rdma.wait()


def _xchg_gather_body(x_ref, slot_ref, out_ref, comm_ref, sems):
    my_x = lax.axis_index("x")
    my_y = lax.axis_index("y")
    peer = (my_x, 1 - my_y)

    barrier = pltpu.get_barrier_semaphore()
    pl.semaphore_signal(
        barrier, inc=1, device_id=peer, device_id_type=pl.DeviceIdType.MESH
    )
    pl.semaphore_wait(barrier, 1)

    rdma = pltpu.make_async_remote_copy(
        src_ref=x_ref,
        dst_ref=comm_ref,
        send_sem=sems.at[0],
        recv_sem=sems.at[1],
        device_id=peer,
        device_id_type=pl.DeviceIdType.MESH,
    )
    rdma.start()

    out_ref[...] = jnp.zeros_like(out_ref)

    def gather_from(buf_ref, tok_base):
        def kk_body(kk, _):
            ids = slot_ref[:, pl.ds(tok_base + kk * K_TILE, K_TILE)]
            xblk = buf_ref[pl.ds(kk * K_TILE, K_TILE), :]

            def s_body(s, _):
                iota = jax.lax.broadcasted_iota(
                    jnp.int32, (S_TILE, K_TILE), 0
                )
                p = (iota + s * S_TILE == ids).astype(jnp.bfloat16)
                contrib = jnp.dot(
                    p, xblk, preferred_element_type=jnp.float32
                ).astype(jnp.bfloat16)
                out_ref[pl.ds(s * S_TILE, S_TILE), :] += contrib
                return 0

            return lax.fori_loop(0, N_SLOTS // S_TILE, s_body, 0)

        lax.fori_loop(0, T_LOCAL // K_TILE, kk_body, 0)

    gather_from(x_ref, my_y * T_LOCAL)
    rdma.wait()
    gather_from(comm_ref, (1 - my_y) * T_LOCAL)


def _scatter_mm_body(slot_ref, y_ref, out_ref):
    s = pl.program_id(1)
    iota = jax.lax.broadcasted_iota(jnp.int32, (K_TILE, S_TILE), 1)
    p = (slot_ref[...] == iota + s * S_TILE).astype(jnp.bfloat16)
    contrib = jnp.dot(
        p, y_ref[...], preferred_element_type=jnp.float32
    ).astype(jnp.bfloat16)

    @pl.when(s == 0)
    def _():
        out_ref[...] = contrib

    @pl.when(s != 0)
    def _():
        out_ref[...] += contrib


def _moe_body(x_ref, w1_ref, w2_ref, out_ref):
    f = pl.program_id(1)
    w1 = w1_ref[0].astype(jnp.bfloat16)
    w2 = w2_ref[0].astype(jnp.bfloat16)
    h = jnp.maximum(
        jnp.dot(x_ref[...], w1, preferred_element_type=jnp.float32),
        0.0,
    ).astype(jnp.bfloat16)
    o = jnp.dot(h, w2, preferred_element_type=jnp.float32)
    contrib = o.astype(jnp.bfloat16)

    @pl.when(f == 0)
    def _():
        out_ref[...] = contrib

    @pl.when(f != 0)
    def _():
        out_ref[...] += contrib


def _reduce_body(p_ref, out_ref, local_ref, comm_ref, sems):
    my_x = lax.axis_index("x")
    my_y = lax.axis_index("y")
    peer = (my_x, 1 - my_y)

    barrier = pltpu.get_barrier_semaphore()
    pl.semaphore_signal(
        barrier, inc=1, device_id=peer, device_id_type=pl.DeviceIdType.MESH
    )
    pl.semaphore_wait(barrier, 1)

    rdma = pltpu.make_async_remote_copy(
        src_ref=p_ref.at[pl.ds((1 - my_y) * T_LOCAL, T_LOCAL), :],
        dst_ref=comm_ref,
        send_sem=sems.at[0],
        recv_sem=sems.at[1],
        device_id=peer,
        device_id_type=pl.DeviceIdType.MESH,
    )
    rdma.start()

    local_copy = pltpu.make_async_copy(
        p_ref.at[pl.ds(my_y * T_LOCAL, T_LOCAL), :], local_ref, sems.at[2]
    )
    local_copy.start()
    local_copy.wait()
    rdma.wait()

    out_ref[...] = local_ref[...] + comm_ref[...]


def kernel(x, assign, W1, W2):
    xb = x.astype(jnp.bfloat16)
    a2 = assign.reshape(1, T_LOCAL)

    aall = pl.pallas_call(
        _assign_xchg_body,
        out_shape=jax.ShapeDtypeStruct((2, T_LOCAL), jnp.int32),
        in_specs=[pl.BlockSpec(memory_space=pltpu.VMEM)],
        out_specs=pl.BlockSpec(memory_space=pltpu.VMEM),
        scratch_shapes=[pltpu.SemaphoreType.DMA((2,))],
        compiler_params=pltpu.CompilerParams(collective_id=0),
    )(a2)

    my_y = lax.axis_index("y")
    t_all = 2 * T_LOCAL
    a_flat = aall.reshape(t_all)
    onehot = (
        a_flat[:, None] == jnp.arange(8, dtype=jnp.int32)[None, :]
    ).astype(jnp.int32)
    rank = jnp.sum((jnp.cumsum(onehot, axis=0) - onehot) * onehot, axis=1)
    slot_tok = a_flat * CAP + rank - 4 * my_y * CAP

    n_slots = N_SLOTS
    x_routed = pl.pallas_call(
        _xchg_gather_body,
        out_shape=jax.ShapeDtypeStruct((n_slots, D), jnp.bfloat16),
        in_specs=[
            pl.BlockSpec(memory_space=pltpu.VMEM),
            pl.BlockSpec(memory_space=pltpu.VMEM),
        ],
        out_specs=pl.BlockSpec(memory_space=pltpu.VMEM),
        scratch_shapes=[
            pltpu.VMEM((T_LOCAL, D), jnp.bfloat16),
            pltpu.SemaphoreType.DMA((2,)),
        ],
        compiler_params=pltpu.CompilerParams(
            collective_id=1,
            vmem_limit_bytes=60 * 1024 * 1024,
        ),
    )(xb, slot_tok.reshape(1, t_all))

    n_f = F // TILE_F
    y_routed = pl.pallas_call(
        _moe_body,
        grid=(E_LOCAL, n_f),
        in_specs=[
            pl.BlockSpec((CAP, D), lambda e, f: (e, 0)),
            pl.BlockSpec((1, D, TILE_F), lambda e, f: (e, 0, f)),
            pl.BlockSpec((1, TILE_F, D), lambda e, f: (e, f, 0)),
        ],
        out_specs=pl.BlockSpec((CAP, D), lambda e, f: (e, 0)),
        out_shape=jax.ShapeDtypeStruct((E_LOCAL * CAP, D), jnp.bfloat16),
        compiler_params=pltpu.CompilerParams(
            dimension_semantics=("arbitrary", "arbitrary"),
            vmem_limit_bytes=60 * 1024 * 1024,
        ),
    )(x_routed, W1, W2)

    partial = pl.pallas_call(
        _scatter_mm_body,
        grid=(t_all // K_TILE, n_slots // S_TILE),
        in_specs=[
            pl.BlockSpec((K_TILE, 1), lambda t, s: (t, 0)),
            pl.BlockSpec((S_TILE, D), lambda t, s: (s, 0)),
        ],
        out_specs=pl.BlockSpec((K_TILE, D), lambda t, s: (t, 0)),
        out_shape=jax.ShapeDtypeStruct((t_all, D), jnp.bfloat16),
        compiler_params=pltpu.CompilerParams(
            dimension_semantics=("arbitrary", "arbitrary"),
        ),
    )(slot_tok.reshape(t_all, 1), y_routed)

    out = pl.pallas_call(
        _reduce_body,
        out_shape=jax.ShapeDtypeStruct((T_LOCAL, D), jnp.bfloat16),
        in_specs=[pl.BlockSpec(memory_space=pl.ANY)],
        out_specs=pl.BlockSpec(memory_space=pltpu.VMEM),
        scratch_shapes=[
            pltpu.VMEM((T_LOCAL, D), jnp.bfloat16),
            pltpu.VMEM((T_LOCAL, D), jnp.bfloat16),
            pltpu.SemaphoreType.DMA((3,)),
        ],
        compiler_params=pltpu.CompilerParams(collective_id=2),
    )(partial)
    return out.astype(jnp.float32)


# device time: 830428 ns/iter; 1.1659x vs baseline; 1.1659x over previous
import jax
import jax.numpy as jnp
from jax import lax
from jax.experimental import pallas as pl
from jax.experimental.pallas import tpu as pltpu

T_LOCAL = 4096
D = 2048
F = 4096
E_LOCAL = 4
TILE_F = 512
CAP = 1152
S_TILE = 512
K_TILE = 512
N_SLOTS = E_LOCAL * CAP


def _assign_xchg_body(a_ref, aall_ref, sems):
    my_x = lax.axis_index("x")
    my_y = lax.axis_index("y")
    peer = (my_x, 1 - my_y)

    barrier = pltpu.get_barrier_semaphore()
    pl.semaphore_signal(
        barrier, inc=1, device_id=peer, device_id_type=pl.DeviceIdType.MESH
    )
    pl.semaphore_wait(barrier, 1)

    aall_ref[pl.ds(my_y, 1), :] = a_ref[...]
    rdma = pltpu.make_async_remote_copy(
        src_ref=a_ref,
        dst_ref=aall_ref.at[pl.ds(my_y, 1), :],
        send_sem=sems.at[0],
        recv_sem=sems.at[1],
        device_id=peer,
        device_id_type=pl.DeviceIdType.MESH,
    )
    rdma.start()
    rdma.wait()


def _xchg_gather_body(x_ref, slot_ref, out_ref, comm_ref, sems):
    my_x = lax.axis_index("x")
    my_y = lax.axis_index("y")
    peer = (my_x, 1 - my_y)

    barrier = pltpu.get_barrier_semaphore()
    pl.semaphore_signal(
        barrier, inc=1, device_id=peer, device_id_type=pl.DeviceIdType.MESH
    )
    pl.semaphore_wait(barrier, 1)

    rdma = pltpu.make_async_remote_copy(
        src_ref=x_ref,
        dst_ref=comm_ref,
        send_sem=sems.at[0],
        recv_sem=sems.at[1],
        device_id=peer,
        device_id_type=pl.DeviceIdType.MESH,
    )
    rdma.start()

    out_ref[...] = jnp.zeros_like(out_ref)

    def gather_from(buf_ref, tok_base):
        def kk_body(kk, _):
            ids = slot_ref[:, pl.ds(tok_base + kk * K_TILE, K_TILE)]
            xblk = buf_ref[pl.ds(kk * K_TILE, K_TILE), :]

            def s_body(s, _):
                iota = jax.lax.broadcasted_iota(
                    jnp.int32, (S_TILE, K_TILE), 0
                )
                p = (iota + s * S_TILE == ids).astype(jnp.bfloat16)
                contrib = jnp.dot(
                    p, xblk, preferred_element_type=jnp.float32
                ).astype(jnp.bfloat16)
                out_ref[pl.ds(s * S_TILE, S_TILE), :] += contrib
                return 0

            return lax.fori_loop(0, N_SLOTS // S_TILE, s_body, 0)

        lax.fori_loop(0, T_LOCAL // K_TILE, kk_body, 0)

    gather_from(x_ref, my_y * T_LOCAL)
    rdma.wait()
    gather_from(comm_ref, (1 - my_y) * T_LOCAL)


def _scatter_reduce_body(slot_ref, y_ref, out_ref, comm_ref, peerbuf_ref, sems):
    my_x = lax.axis_index("x")
    my_y = lax.axis_index("y")
    peer = (my_x, 1 - my_y)

    barrier = pltpu.get_barrier_semaphore()
    pl.semaphore_signal(
        barrier, inc=1, device_id=peer, device_id_type=pl.DeviceIdType.MESH
    )
    pl.semaphore_wait(barrier, 1)

    def scatter_half(dst_ref, tok_base):
        dst_ref[...] = jnp.zeros_like(dst_ref)

        def tt_body(tt, _):
            ids = slot_ref[:, pl.ds(tok_base + tt * K_TILE, K_TILE)]

            def s_body(s, _):
                iota = jax.lax.broadcasted_iota(
                    jnp.int32, (S_TILE, K_TILE), 0
                )
                pt = (iota + s * S_TILE == ids).astype(jnp.bfloat16)
                yblk = y_ref[pl.ds(s * S_TILE, S_TILE), :]
                contrib = jax.lax.dot_general(
                    pt,
                    yblk,
                    (((0,), (0,)), ((), ())),
                    preferred_element_type=jnp.float32,
                ).astype(jnp.bfloat16)
                dst_ref[pl.ds(tt * K_TILE, K_TILE), :] += contrib
                return 0

            return lax.fori_loop(0, N_SLOTS // S_TILE, s_body, 0)

        lax.fori_loop(0, T_LOCAL // K_TILE, tt_body, 0)

    scatter_half(peerbuf_ref, (1 - my_y) * T_LOCAL)
    rdma = pltpu.make_async_remote_copy(
        src_ref=peerbuf_ref,
        dst_ref=comm_ref,
        send_sem=sems.at[0],
        recv_sem=sems.at[1],
        device_id=peer,
        device_id_type=pl.DeviceIdType.MESH,
    )
    rdma.start()
    scatter_half(out_ref, my_y * T_LOCAL)
    rdma.wait()
    copy = pltpu.make_async_copy(comm_ref, peerbuf_ref, sems.at[2])
    copy.start()
    copy.wait()
    out_ref[...] += peerbuf_ref[...]


def _moe_body(x_ref, w1_ref, w2_ref, out_ref):
    f = pl.program_id(1)
    w1 = w1_ref[0].astype(jnp.bfloat16)
    w2 = w2_ref[0].astype(jnp.bfloat16)
    h = jnp.maximum(
        jnp.dot(x_ref[...], w1, preferred_element_type=jnp.float32),
        0.0,
    ).astype(jnp.bfloat16)
    o = jnp.dot(h, w2, preferred_element_type=jnp.float32)
    contrib = o.astype(jnp.bfloat16)

    @pl.when(f == 0)
    def _():
        out_ref[...] = contrib

    @pl.when(f != 0)
    def _():
        out_ref[...] += contrib


def kernel(x, assign, W1, W2):
    xb = x.astype(jnp.bfloat16)
    a2 = assign.reshape(1, T_LOCAL)

    aall = pl.pallas_call(
        _assign_xchg_body,
        out_shape=jax.ShapeDtypeStruct((2, T_LOCAL), jnp.int32),
        in_specs=[pl.BlockSpec(memory_space=pltpu.VMEM)],
        out_specs=pl.BlockSpec(memory_space=pltpu.VMEM),
        scratch_shapes=[pltpu.SemaphoreType.DMA((2,))],
        compiler_params=pltpu.CompilerParams(collective_id=0),
    )(a2)

    my_y = lax.axis_index("y")
    t_all = 2 * T_LOCAL
    a_flat = aall.reshape(t_all)
    onehot = (
        a_flat[:, None] == jnp.arange(8, dtype=jnp.int32)[None, :]
    ).astype(jnp.int32)
    rank = jnp.sum((jnp.cumsum(onehot, axis=0) - onehot) * onehot, axis=1)
    slot_tok = a_flat * CAP + rank - 4 * my_y * CAP

    n_slots = N_SLOTS
    x_routed = pl.pallas_call(
        _xchg_gather_body,
        out_shape=jax.ShapeDtypeStruct((n_slots, D), jnp.bfloat16),
        in_specs=[
            pl.BlockSpec(memory_space=pltpu.VMEM),
            pl.BlockSpec(memory_space=pltpu.VMEM),
        ],
        out_specs=pl.BlockSpec(memory_space=pltpu.VMEM),
        scratch_shapes=[
            pltpu.VMEM((T_LOCAL, D), jnp.bfloat16),
            pltpu.SemaphoreType.DMA((2,)),
        ],
        compiler_params=pltpu.CompilerParams(
            collective_id=1,
            vmem_limit_bytes=60 * 1024 * 1024,
        ),
    )(xb, slot_tok.reshape(1, t_all))

    n_f = F // TILE_F
    y_routed = pl.pallas_call(
        _moe_body,
        grid=(E_LOCAL, n_f),
        in_specs=[
            pl.BlockSpec((CAP, D), lambda e, f: (e, 0)),
            pl.BlockSpec((1, D, TILE_F), lambda e, f: (e, 0, f)),
            pl.BlockSpec((1, TILE_F, D), lambda e, f: (e, f, 0)),
        ],
        out_specs=pl.BlockSpec((CAP, D), lambda e, f: (e, 0)),
        out_shape=jax.ShapeDtypeStruct((E_LOCAL * CAP, D), jnp.bfloat16),
        compiler_params=pltpu.CompilerParams(
            dimension_semantics=("arbitrary", "arbitrary"),
            vmem_limit_bytes=60 * 1024 * 1024,
        ),
    )(x_routed, W1, W2)

    out, _ = pl.pallas_call(
        _scatter_reduce_body,
        out_shape=(
            jax.ShapeDtypeStruct((T_LOCAL, D), jnp.bfloat16),
            jax.ShapeDtypeStruct((T_LOCAL, D), jnp.bfloat16),
        ),
        in_specs=[
            pl.BlockSpec(memory_space=pltpu.VMEM),
            pl.BlockSpec(memory_space=pltpu.VMEM),
        ],
        out_specs=(
            pl.BlockSpec(memory_space=pltpu.VMEM),
            pl.BlockSpec(memory_space=pl.ANY),
        ),
        scratch_shapes=[
            pltpu.VMEM((T_LOCAL, D), jnp.bfloat16),
            pltpu.SemaphoreType.DMA((3,)),
        ],
        compiler_params=pltpu.CompilerParams(
            collective_id=2,
            vmem_limit_bytes=60 * 1024 * 1024,
        ),
    )(slot_tok.reshape(1, t_all), y_routed)
    return out.astype(jnp.float32)


# device time: 736767 ns/iter; 1.3141x vs baseline; 1.1271x over previous
import jax
import jax.numpy as jnp
from jax import lax
from jax.experimental import pallas as pl
from jax.experimental.pallas import tpu as pltpu

T_LOCAL = 4096
D = 2048
F = 4096
E_LOCAL = 4
TILE_F = 512
CAP = 1152
S_TILE = 512
K_TILE = 512
N_SLOTS = E_LOCAL * CAP


def _assign_xchg_body(a_ref, aall_ref, sems):
    my_x = lax.axis_index("x")
    my_y = lax.axis_index("y")
    peer = (my_x, 1 - my_y)

    barrier = pltpu.get_barrier_semaphore()
    pl.semaphore_signal(
        barrier, inc=1, device_id=peer, device_id_type=pl.DeviceIdType.MESH
    )
    pl.semaphore_wait(barrier, 1)

    aall_ref[pl.ds(my_y, 1), :] = a_ref[...]
    rdma = pltpu.make_async_remote_copy(
        src_ref=a_ref,
        dst_ref=aall_ref.at[pl.ds(my_y, 1), :],
        send_sem=sems.at[0],
        recv_sem=sems.at[1],
        device_id=peer,
        device_id_type=pl.DeviceIdType.MESH,
    )
    rdma.start()
    rdma.wait()


def _xchg_gather_body(x_ref, slot_ref, out_ref, comm_ref, sems):
    my_x = lax.axis_index("x")
    my_y = lax.axis_index("y")
    peer = (my_x, 1 - my_y)

    barrier = pltpu.get_barrier_semaphore()
    pl.semaphore_signal(
        barrier, inc=1, device_id=peer, device_id_type=pl.DeviceIdType.MESH
    )
    pl.semaphore_wait(barrier, 1)

    half = T_LOCAL // 2
    rdmas = []
    for c in range(2):
        rdmas.append(
            pltpu.make_async_remote_copy(
                src_ref=x_ref.at[pl.ds(c * half, half), :],
                dst_ref=comm_ref.at[pl.ds(c * half, half), :],
                send_sem=sems.at[2 * c],
                recv_sem=sems.at[2 * c + 1],
                device_id=peer,
                device_id_type=pl.DeviceIdType.MESH,
            )
        )
        rdmas[c].start()

    out_ref[...] = jnp.zeros_like(out_ref)

    def gather_from(buf_ref, tok_base, kk_lo, kk_hi):
        def kk_body(kk, _):
            ids = slot_ref[:, pl.ds(tok_base + kk * K_TILE, K_TILE)]
            xblk = buf_ref[pl.ds(kk * K_TILE, K_TILE), :]

            def s_body(s, _):
                iota = jax.lax.broadcasted_iota(
                    jnp.int32, (S_TILE, K_TILE), 0
                )
                p = (iota + s * S_TILE == ids).astype(jnp.bfloat16)
                contrib = jnp.dot(
                    p, xblk, preferred_element_type=jnp.float32
                ).astype(jnp.bfloat16)
                out_ref[pl.ds(s * S_TILE, S_TILE), :] += contrib
                return 0

            return lax.fori_loop(0, N_SLOTS // S_TILE, s_body, 0)

        lax.fori_loop(kk_lo, kk_hi, kk_body, 0)

    n_kk = T_LOCAL // K_TILE
    gather_from(x_ref, my_y * T_LOCAL, 0, n_kk)
    peer_base = (1 - my_y) * T_LOCAL
    rdmas[0].wait()
    gather_from(comm_ref, peer_base, 0, n_kk // 2)
    rdmas[1].wait()
    gather_from(comm_ref, peer_base, n_kk // 2, n_kk)


def _scatter_reduce_body(slot_ref, y_ref, out_ref, comm_ref, peerbuf_ref, sems):
    my_x = lax.axis_index("x")
    my_y = lax.axis_index("y")
    peer = (my_x, 1 - my_y)

    barrier = pltpu.get_barrier_semaphore()
    pl.semaphore_signal(
        barrier, inc=1, device_id=peer, device_id_type=pl.DeviceIdType.MESH
    )
    pl.semaphore_wait(barrier, 1)

    def scatter_range(dst_ref, tok_base, tt_lo, tt_hi):
        dst_ref[pl.ds(tt_lo * K_TILE, (tt_hi - tt_lo) * K_TILE), :] = (
            jnp.zeros(((tt_hi - tt_lo) * K_TILE, D), jnp.bfloat16)
        )

        def tt_body(tt, _):
            ids = slot_ref[:, pl.ds(tok_base + tt * K_TILE, K_TILE)]

            def s_body(s, _):
                iota = jax.lax.broadcasted_iota(
                    jnp.int32, (S_TILE, K_TILE), 0
                )
                pt = (iota + s * S_TILE == ids).astype(jnp.bfloat16)
                yblk = y_ref[pl.ds(s * S_TILE, S_TILE), :]
                contrib = jax.lax.dot_general(
                    pt,
                    yblk,
                    (((0,), (0,)), ((), ())),
                    preferred_element_type=jnp.float32,
                ).astype(jnp.bfloat16)
                dst_ref[pl.ds(tt * K_TILE, K_TILE), :] += contrib
                return 0

            return lax.fori_loop(0, N_SLOTS // S_TILE, s_body, 0)

        lax.fori_loop(tt_lo, tt_hi, tt_body, 0)

    n_tt = T_LOCAL // K_TILE
    half = T_LOCAL // 2
    peer_base = (1 - my_y) * T_LOCAL
    rdmas = []
    for c in range(2):
        scatter_range(peerbuf_ref, peer_base, c * n_tt // 2, (c + 1) * n_tt // 2)
        rdmas.append(
            pltpu.make_async_remote_copy(
                src_ref=peerbuf_ref.at[pl.ds(c * half, half), :],
                dst_ref=comm_ref.at[pl.ds(c * half, half), :],
                send_sem=sems.at[2 * c],
                recv_sem=sems.at[2 * c + 1],
                device_id=peer,
                device_id_type=pl.DeviceIdType.MESH,
            )
        )
        rdmas[c].start()
    scatter_range(out_ref, my_y * T_LOCAL, 0, n_tt)
    rdmas[0].wait()
    rdmas[1].wait()
    copy = pltpu.make_async_copy(comm_ref, peerbuf_ref, sems.at[4])
    copy.start()
    copy.wait()
    out_ref[...] += peerbuf_ref[...]


def _moe_body(x_ref, w1_ref, w2_ref, out_ref):
    f = pl.program_id(1)
    w1 = w1_ref[0].astype(jnp.bfloat16)
    w2 = w2_ref[0].astype(jnp.bfloat16)
    h = jnp.maximum(
        jnp.dot(x_ref[...], w1, preferred_element_type=jnp.float32),
        0.0,
    ).astype(jnp.bfloat16)
    o = jnp.dot(h, w2, preferred_element_type=jnp.float32)
    contrib = o.astype(jnp.bfloat16)

    @pl.when(f == 0)
    def _():
        out_ref[...] = contrib

    @pl.when(f != 0)
    def _():
        out_ref[...] += contrib


def kernel(x, assign, W1, W2):
    xb = x.astype(jnp.bfloat16)
    a2 = assign.reshape(1, T_LOCAL)

    aall = pl.pallas_call(
        _assign_xchg_body,
        out_shape=jax.ShapeDtypeStruct((2, T_LOCAL), jnp.int32),
        in_specs=[pl.BlockSpec(memory_space=pltpu.VMEM)],
        out_specs=pl.BlockSpec(memory_space=pltpu.VMEM),
        scratch_shapes=[pltpu.SemaphoreType.DMA((2,))],
        compiler_params=pltpu.CompilerParams(collective_id=0),
    )(a2)

    my_y = lax.axis_index("y")
    t_all = 2 * T_LOCAL
    a_flat = aall.reshape(t_all)
    onehot = (
        a_flat[:, None] == jnp.arange(8, dtype=jnp.int32)[None, :]
    ).astype(jnp.int32)
    rank = jnp.sum((jnp.cumsum(onehot, axis=0) - onehot) * onehot, axis=1)
    slot_tok = a_flat * CAP + rank - 4 * my_y * CAP

    n_slots = N_SLOTS
    x_routed = pl.pallas_call(
        _xchg_gather_body,
        out_shape=jax.ShapeDtypeStruct((n_slots, D), jnp.bfloat16),
        in_specs=[
            pl.BlockSpec(memory_space=pltpu.VMEM),
            pl.BlockSpec(memory_space=pltpu.VMEM),
        ],
        out_specs=pl.BlockSpec(memory_space=pltpu.VMEM),
        scratch_shapes=[
            pltpu.VMEM((T_LOCAL, D), jnp.bfloat16),
            pltpu.SemaphoreType.DMA((4,)),
        ],
        compiler_params=pltpu.CompilerParams(
            collective_id=1,
            vmem_limit_bytes=60 * 1024 * 1024,
        ),
    )(xb, slot_tok.reshape(1, t_all))

    n_f = F // TILE_F
    y_routed = pl.pallas_call(
        _moe_body,
        grid=(E_LOCAL, n_f),
        in_specs=[
            pl.BlockSpec((CAP, D), lambda e, f: (e, 0)),
            pl.BlockSpec((1, D, TILE_F), lambda e, f: (e, 0, f)),
            pl.BlockSpec((1, TILE_F, D), lambda e, f: (e, f, 0)),
        ],
        out_specs=pl.BlockSpec((CAP, D), lambda e, f: (e, 0)),
        out_shape=jax.ShapeDtypeStruct((E_LOCAL * CAP, D), jnp.bfloat16),
        compiler_params=pltpu.CompilerParams(
            dimension_semantics=("arbitrary", "arbitrary"),
            vmem_limit_bytes=60 * 1024 * 1024,
        ),
    )(x_routed, W1, W2)

    out, _ = pl.pallas_call(
        _scatter_reduce_body,
        out_shape=(
            jax.ShapeDtypeStruct((T_LOCAL, D), jnp.bfloat16),
            jax.ShapeDtypeStruct((T_LOCAL, D), jnp.bfloat16),
        ),
        in_specs=[
            pl.BlockSpec(memory_space=pltpu.VMEM),
            pl.BlockSpec(memory_space=pltpu.VMEM),
        ],
        out_specs=(
            pl.BlockSpec(memory_space=pltpu.VMEM),
            pl.BlockSpec(memory_space=pl.ANY),
        ),
        scratch_shapes=[
            pltpu.VMEM((T_LOCAL, D), jnp.bfloat16),
            pltpu.SemaphoreType.DMA((5,)),
        ],
        compiler_params=pltpu.CompilerParams(
            collective_id=2,
            vmem_limit_bytes=60 * 1024 * 1024,
        ),
    )(slot_tok.reshape(1, t_all), y_routed)
    return out.astype(jnp.float32)
